# baseline (speedup 1.0000x reference)
import jax
import jax.numpy as jnp
from jax.experimental import pallas as pl
from jax.experimental.pallas import tpu as pltpu

_NS = [778, 389, 195, 98, 49]
_SEQ = 9
_LAT = 128
_BBLK = 256
_U = 16
_C = [16, 32, 64, 64]
_COUT = [32, 64, 64, 64]


def _elu(x):
    return jnp.where(x > 0, x, jnp.exp(jnp.minimum(x, 0.0)) - 1.0)


def _gather(sp_ref, src_ref, g_ref, p, j, n_in, c_in):
    n0 = _U * j
    for q in range(_U):
        nq = jnp.minimum(n0 + q, n_in - 1)
        for s in range(_SEQ):
            off = sp_ref[nq, s] * c_in
            g_ref[p, pl.ds(s * c_in, c_in), q * _BBLK:(q + 1) * _BBLK] = (
                src_ref[pl.ds(off, c_in), :])


def _mm(w_ref, g_ref, r_ref, p, c_in, c_out):
    gg = g_ref[p, pl.ds(0, _SEQ * c_in), :]
    r_ref[p, 0:c_out, :] = jnp.dot(w_ref[...], gg,
                                   preferred_element_type=jnp.float32)


def _finish(b_ref, r_ref, dst_ref, p, j, c_out):
    res = _elu(r_ref[p, 0:c_out, :] + b_ref[...])
    parts = [0.5 * (res[:, (2 * k) * _BBLK:(2 * k + 1) * _BBLK] +
                    res[:, (2 * k + 1) * _BBLK:(2 * k + 2) * _BBLK])
             for k in range(_U // 2)]
    pooled = jnp.concatenate(parts, axis=0).astype(jnp.bfloat16)
    row = jnp.maximum(j, 0) * (_U // 2) * c_out
    dst_ref[pl.ds(row, (_U // 2) * c_out), :] = pooled


def _level(sp_ref, src_ref, w_ref, b_ref, dst_ref, g_ref, r_ref,
           n_in, c_in, c_out):
    n_out = (n_in + 1) // 2
    m = (n_out + _U // 2 - 1) // (_U // 2)

    _gather(sp_ref, src_ref, g_ref, 0, 0, n_in, c_in)

    def body(j, _):
        p = jax.lax.bitwise_and(j, 1)
        _finish(b_ref, r_ref, dst_ref, 1 - p, j - 1, c_out)
        _mm(w_ref, g_ref, r_ref, p, c_in, c_out)
        _gather(sp_ref, src_ref, g_ref, 1 - p, j + 1, n_in, c_in)
        return 0

    jax.lax.fori_loop(0, m, body, 0)
    _finish(b_ref, r_ref, dst_ref, jax.lax.bitwise_and(m - 1, 1), m - 1, c_out)


def _fused_kernel(sp0, sp1, sp2, sp3,
                  x_ref, w0, b0, w1, b1, w2, b2, w3, b3, wl, bl,
                  out_ref,
                  h1, h2, h3, h4, g, r):
    _level(sp0, x_ref, w0, b0, h1, g, r, _NS[0], _C[0], _COUT[0])
    _level(sp1, h1, w1, b1, h2, g, r, _NS[1], _C[1], _COUT[1])
    _level(sp2, h2, w2, b2, h3, g, r, _NS[2], _C[2], _COUT[2])
    _level(sp3, h3, w3, b3, h4, g, r, _NS[3], _C[3], _COUT[3])
    hf = h4[pl.ds(0, _NS[4] * _COUT[3]), :]
    out_ref[...] = jnp.dot(wl[...], hf, preferred_element_type=jnp.float32) + bl[...]


def kernel(x, sp0, sp1, sp2, sp3, dt0, dt1, dt2, dt3,
           W0, b0, W1, b1, W2, b2, W3, b3, Wl, bl):
    T, B = x.shape[0], x.shape[1]
    BT = T * B
    xt = jnp.transpose(x.reshape(BT, _NS[0], 3), (1, 2, 0))
    xp = jnp.pad(xt, ((0, 0), (0, _C[0] - 3), (0, 0)))
    xp = xp.reshape(_NS[0] * _C[0], BT).astype(jnp.bfloat16)
    W0p = jnp.pad(W0.reshape(_COUT[0], _SEQ, 3), ((0, 0), (0, 0), (0, _C[0] - 3)))
    W0p = W0p.reshape(_COUT[0], _SEQ * _C[0])

    bf = lambda w: w.astype(jnp.bfloat16)
    col = lambda v: v[:, None]
    nblocks = BT // _BBLK
    h_rows = [(_NS[i + 1] + _U) * _COUT[i] for i in range(4)]

    full = lambda shape: pl.BlockSpec(shape, lambda i, *_: (0, 0))
    out = pl.pallas_call(
        _fused_kernel,
        grid_spec=pltpu.PrefetchScalarGridSpec(
            num_scalar_prefetch=4,
            grid=(nblocks,),
            in_specs=[
                pl.BlockSpec((_NS[0] * _C[0], _BBLK), lambda i, *_: (0, i)),
                full(W0p.shape), full((_COUT[0], 1)),
                full(W1.shape), full((_COUT[1], 1)),
                full(W2.shape), full((_COUT[2], 1)),
                full(W3.shape), full((_COUT[3], 1)),
                full(Wl.shape), full((_LAT, 1)),
            ],
            out_specs=pl.BlockSpec((_LAT, _BBLK), lambda i, *_: (0, i)),
            scratch_shapes=[
                pltpu.VMEM((h_rows[0], _BBLK), jnp.bfloat16),
                pltpu.VMEM((h_rows[1], _BBLK), jnp.bfloat16),
                pltpu.VMEM((h_rows[2], _BBLK), jnp.bfloat16),
                pltpu.VMEM((h_rows[3], _BBLK), jnp.bfloat16),
                pltpu.VMEM((2, _SEQ * 64, _U * _BBLK), jnp.bfloat16),
                pltpu.VMEM((2, 64, _U * _BBLK), jnp.float32),
            ],
        ),
        out_shape=jax.ShapeDtypeStruct((_LAT, BT), jnp.float32),
    )(sp0, sp1, sp2, sp3,
      xp, bf(W0p), col(b0), bf(W1), col(b1), bf(W2), col(b2), bf(W3), col(b3),
      bf(Wl), col(bl))
    return out.T.reshape(T, B, _LAT)


# speedup vs baseline: 1.2839x; 1.2839x over previous
import jax
import jax.numpy as jnp
from jax.experimental import pallas as pl
from jax.experimental.pallas import tpu as pltpu

_NS = [778, 389, 195, 98, 49]
_SEQ = 9
_LAT = 128
_BBLK = 256
_U = 8
_C = [16, 32, 64, 64]
_COUT = [32, 64, 64, 64]


def _elu(x):
    return jnp.where(x > 0, x, jnp.exp(jnp.minimum(x, 0.0)) - 1.0)


def _gather(sp_ref, src_ref, g_ref, p, j, n_in, c_in):
    n0 = _U * j
    for q in range(_U):
        nq = jnp.minimum(n0 + q, n_in - 1)
        for s in range(_SEQ):
            off = sp_ref[nq, s] * c_in
            g_ref[p, pl.ds(s * c_in, c_in), q * _BBLK:(q + 1) * _BBLK] = (
                src_ref[pl.ds(off, c_in), :])


def _mm(w_ref, g_ref, r_ref, p, c_in, c_out):
    gg = g_ref[p, pl.ds(0, _SEQ * c_in), :]
    r_ref[p, 0:c_out, :] = jnp.dot(w_ref[...], gg,
                                   preferred_element_type=jnp.float32).astype(jnp.bfloat16)


def _finish(b_ref, r_ref, dst_ref, p, j, c_out):
    res = _elu(r_ref[p, 0:c_out, :].astype(jnp.float32) + b_ref[...])
    parts = [0.5 * (res[:, (2 * k) * _BBLK:(2 * k + 1) * _BBLK] +
                    res[:, (2 * k + 1) * _BBLK:(2 * k + 2) * _BBLK])
             for k in range(_U // 2)]
    pooled = jnp.concatenate(parts, axis=0).astype(jnp.bfloat16)
    row = jnp.maximum(j, 0) * (_U // 2) * c_out
    dst_ref[pl.ds(row, (_U // 2) * c_out), :] = pooled


def _level(sp_ref, src_ref, w_ref, b_ref, dst_ref, g_ref, r_ref,
           n_in, c_in, c_out):
    n_out = (n_in + 1) // 2
    m = (n_out + _U // 2 - 1) // (_U // 2)

    _gather(sp_ref, src_ref, g_ref, 0, 0, n_in, c_in)

    def body(j, _):
        p = jax.lax.bitwise_and(j, 1)
        _finish(b_ref, r_ref, dst_ref, 1 - p, j - 1, c_out)
        _mm(w_ref, g_ref, r_ref, p, c_in, c_out)
        _gather(sp_ref, src_ref, g_ref, 1 - p, j + 1, n_in, c_in)
        return 0

    jax.lax.fori_loop(0, m, body, 0)
    _finish(b_ref, r_ref, dst_ref, jax.lax.bitwise_and(m - 1, 1), m - 1, c_out)


def _fused_kernel(sp0, sp1, sp2, sp3,
                  x_ref, w0, b0, w1, b1, w2, b2, w3, b3, wl, bl,
                  out_ref,
                  h1, h2, h3, h4, g, r):
    _level(sp0, x_ref, w0, b0, h1, g, r, _NS[0], _C[0], _COUT[0])
    _level(sp1, h1, w1, b1, h2, g, r, _NS[1], _C[1], _COUT[1])
    _level(sp2, h2, w2, b2, h3, g, r, _NS[2], _C[2], _COUT[2])
    _level(sp3, h3, w3, b3, h4, g, r, _NS[3], _C[3], _COUT[3])
    hf = h4[pl.ds(0, _NS[4] * _COUT[3]), :]
    out_ref[...] = jnp.dot(wl[...], hf, preferred_element_type=jnp.float32) + bl[...]


def kernel(x, sp0, sp1, sp2, sp3, dt0, dt1, dt2, dt3,
           W0, b0, W1, b1, W2, b2, W3, b3, Wl, bl):
    T, B = x.shape[0], x.shape[1]
    BT = T * B
    xt = jnp.transpose(x.reshape(BT, _NS[0], 3), (1, 2, 0))
    xp = jnp.pad(xt, ((0, 0), (0, _C[0] - 3), (0, 0)))
    xp = xp.reshape(_NS[0] * _C[0], BT).astype(jnp.bfloat16)
    W0p = jnp.pad(W0.reshape(_COUT[0], _SEQ, 3), ((0, 0), (0, 0), (0, _C[0] - 3)))
    W0p = W0p.reshape(_COUT[0], _SEQ * _C[0])

    bf = lambda w: w.astype(jnp.bfloat16)
    col = lambda v: v[:, None]
    nblocks = BT // _BBLK
    h_rows = [(_NS[i + 1] + _U) * _COUT[i] for i in range(4)]

    full = lambda shape: pl.BlockSpec(shape, lambda i, *_: (0, 0))
    out = pl.pallas_call(
        _fused_kernel,
        grid_spec=pltpu.PrefetchScalarGridSpec(
            num_scalar_prefetch=4,
            grid=(nblocks,),
            in_specs=[
                pl.BlockSpec((_NS[0] * _C[0], _BBLK), lambda i, *_: (0, i)),
                full(W0p.shape), full((_COUT[0], 1)),
                full(W1.shape), full((_COUT[1], 1)),
                full(W2.shape), full((_COUT[2], 1)),
                full(W3.shape), full((_COUT[3], 1)),
                full(Wl.shape), full((_LAT, 1)),
            ],
            out_specs=pl.BlockSpec((_LAT, _BBLK), lambda i, *_: (0, i)),
            scratch_shapes=[
                pltpu.VMEM((h_rows[0], _BBLK), jnp.bfloat16),
                pltpu.VMEM((h_rows[1], _BBLK), jnp.bfloat16),
                pltpu.VMEM((h_rows[2], _BBLK), jnp.bfloat16),
                pltpu.VMEM((h_rows[3], _BBLK), jnp.bfloat16),
                pltpu.VMEM((2, _SEQ * 64, _U * _BBLK), jnp.bfloat16),
                pltpu.VMEM((2, 64, _U * _BBLK), jnp.bfloat16),
            ],
        ),
        out_shape=jax.ShapeDtypeStruct((_LAT, BT), jnp.float32),
    )(sp0, sp1, sp2, sp3,
      xp, bf(W0p), col(b0), bf(W1), col(b1), bf(W2), col(b2), bf(W3), col(b3),
      bf(Wl), col(bl))
    return out.T.reshape(T, B, _LAT)
